# baseline (speedup 1.0000x reference)
import functools

import jax
import jax.numpy as jnp
from jax import lax
from jax.experimental import pallas as pl
from jax.experimental.pallas import tpu as pltpu
from jax.experimental.pallas import tpu_sc as plsc

_D = 1024
_S = 2048
_NCORE = 2
_NSUB = 16
_RPW = _S // _NSUB
_CH = 32
_NCH = _RPW // _CH

_mesh = plsc.VectorSubcoreMesh(
    core_axis_name="c", subcore_axis_name="s",
    num_cores=_NCORE, num_subcores=_NSUB,
)


@functools.partial(
    pl.kernel,
    out_type=jax.ShapeDtypeStruct((2, _S + 1, _D), jnp.float32),
    mesh=_mesh,
    scratch_types=[
        pltpu.VMEM((_CH, _D), jnp.float32),
        pltpu.VMEM((_CH, _D), jnp.float32),
        pltpu.VMEM((8,), jnp.int32),
        pltpu.VMEM((8,), jnp.int32),
        pltpu.VMEM((8, _D), jnp.float32),
        pltpu.VMEM((8, _D), jnp.float32),
        pltpu.VMEM((1, _D), jnp.float32),
        pltpu.SemaphoreType.DMA,
        pltpu.SemaphoreType.DMA,
        pltpu.SemaphoreType.DMA,
        pltpu.SemaphoreType.DMA,
        pltpu.SemaphoreType.DMA,
    ],
)
def _sc_fused(bos_hbm, emb_hbm, idx_hbm, spe_hbm, pos_hbm, out_hbm,
              bufa, bufb, bi_v, pi_v, spe_v, pos_v, row_v,
              sem_ia, sem_ib, sem_oa, sem_ob, sem_g):
    c = lax.axis_index("c")
    s = lax.axis_index("s")
    r0 = s * _RPW

    bufs = (bufa, bufb)
    sem_in = (sem_ia, sem_ib)
    sem_out = (sem_oa, sem_ob)

    def run(cc):
        def mk_in(t):
            return pltpu.make_async_copy(
                emb_hbm.at[cc, pl.ds(r0 + t * _CH, _CH), :],
                bufs[t % 2], sem_in[t % 2])

        def mk_out(t):
            return pltpu.make_async_copy(
                bufs[t % 2], out_hbm.at[cc, pl.ds(r0 + t * _CH, _CH), :],
                sem_out[t % 2])

        mk_in(0).start()
        mk_in(1).start()

        @pl.when(s == _NSUB - 1)
        def _row():
            pltpu.sync_copy(bos_hbm, bi_v)
            pltpu.sync_copy(idx_hbm, pi_v)
            g1 = pltpu.make_async_copy(spe_hbm.at[bi_v], spe_v, sem_g)
            g1.start()
            g1.wait()
            g2 = pltpu.make_async_copy(pos_hbm.at[pi_v], pos_v, sem_g)
            g2.start()
            g2.wait()
            for j in range(_D // 16):
                sl = pl.ds(16 * j, 16)
                row_v[0, sl] = spe_v[0, sl] + pos_v[0, sl]
            rw = pltpu.make_async_copy(row_v, out_hbm.at[cc, pl.ds(_S, 1), :], sem_g)
            rw.start()
            rw.wait()

        for t in range(_NCH):
            mk_in(t).wait()
            mk_out(t).start()
            mk_out(t).wait()
            if t + 2 < _NCH:
                mk_in(t + 2).start()

    @pl.when(c == 0)
    def _c0():
        run(0)

    @pl.when(c == 1)
    def _c1():
        run(1)


def kernel(bos_token, embeds, idx, speech_emb, pos_emb):
    bos8 = jnp.broadcast_to(bos_token.reshape(1).astype(jnp.int32), (8,))
    idx8 = jnp.broadcast_to(idx.reshape(1).astype(jnp.int32), (8,))
    return _sc_fused(bos8, embeds, idx8, speech_emb, pos_emb)


# speedup vs baseline: 2.0079x; 2.0079x over previous
import functools

import jax
import jax.numpy as jnp
from jax import lax
from jax.experimental import pallas as pl
from jax.experimental.pallas import tpu as pltpu
from jax.experimental.pallas import tpu_sc as plsc

_D = 1024
_S = 2048
_NCORE = 2
_NSUB = 16
_NW = _NCORE * _NSUB
_RPW = _S // _NW
_CH = 16
_NCH = _RPW // _CH

_mesh = plsc.VectorSubcoreMesh(
    core_axis_name="c", subcore_axis_name="s",
    num_cores=_NCORE, num_subcores=_NSUB,
)


@functools.partial(
    pl.kernel,
    out_type=jax.ShapeDtypeStruct((_S + 1, 2, _D), jnp.float32),
    mesh=_mesh,
    scratch_types=[
        pltpu.VMEM((_CH, 2, _D), jnp.float32),
        pltpu.VMEM((_CH, 2, _D), jnp.float32),
        pltpu.VMEM((8,), jnp.int32),
        pltpu.VMEM((8,), jnp.int32),
        pltpu.VMEM((8, _D), jnp.float32),
        pltpu.VMEM((8, _D), jnp.float32),
        pltpu.VMEM((1, 2, _D), jnp.float32),
        pltpu.SemaphoreType.DMA,
        pltpu.SemaphoreType.DMA,
        pltpu.SemaphoreType.DMA,
        pltpu.SemaphoreType.DMA,
        pltpu.SemaphoreType.DMA,
    ],
)
def _sc_fused(bos_hbm, emb_hbm, idx_hbm, spe_hbm, pos_hbm, out_hbm,
              bufa, bufb, bi_v, pi_v, spe_v, pos_v, row_v,
              sem_ia, sem_ib, sem_oa, sem_ob, sem_g):
    c = lax.axis_index("c")
    s = lax.axis_index("s")
    w = s * _NCORE + c
    r0 = w * _RPW

    bufs = (bufa, bufb)
    sem_in = (sem_ia, sem_ib)
    sem_out = (sem_oa, sem_ob)

    def mk_in(t, b):
        return pltpu.make_async_copy(
            emb_hbm.at[b, pl.ds(r0 + t * _CH, _CH), :],
            bufs[t % 2].at[:, b, :], sem_in[t % 2])

    def mk_out(t):
        return pltpu.make_async_copy(
            bufs[t % 2], out_hbm.at[pl.ds(r0 + t * _CH, _CH), :, :],
            sem_out[t % 2])

    mk_in(0, 0).start()
    mk_in(0, 1).start()
    mk_in(1, 0).start()
    mk_in(1, 1).start()

    @pl.when(w == _NW - 1)
    def _row():
        pltpu.sync_copy(bos_hbm, bi_v)
        pltpu.sync_copy(idx_hbm, pi_v)
        g1 = pltpu.make_async_copy(spe_hbm.at[bi_v], spe_v, sem_g)
        g1.start()
        g1.wait()
        g2 = pltpu.make_async_copy(pos_hbm.at[pi_v], pos_v, sem_g)
        g2.start()
        g2.wait()
        for j in range(_D // 16):
            sl = pl.ds(16 * j, 16)
            r = spe_v[0, sl] + pos_v[0, sl]
            row_v[0, 0, sl] = r
            row_v[0, 1, sl] = r
        rw = pltpu.make_async_copy(row_v, out_hbm.at[pl.ds(_S, 1), :, :], sem_g)
        rw.start()
        rw.wait()

    for t in range(_NCH):
        mk_in(t, 0).wait()
        mk_in(t, 1).wait()
        mk_out(t).start()
        mk_out(t).wait()
        if t + 2 < _NCH:
            mk_in(t + 2, 0).start()
            mk_in(t + 2, 1).start()


def kernel(bos_token, embeds, idx, speech_emb, pos_emb):
    bos8 = jnp.broadcast_to(bos_token.reshape(1).astype(jnp.int32), (8,))
    idx8 = jnp.broadcast_to(idx.reshape(1).astype(jnp.int32), (8,))
    out2 = _sc_fused(bos8, embeds, idx8, speech_emb, pos_emb)
    return jnp.swapaxes(out2, 0, 1)


# speedup vs baseline: 2.4061x; 1.1983x over previous
import functools

import jax
import jax.numpy as jnp
from jax import lax
from jax.experimental import pallas as pl
from jax.experimental.pallas import tpu as pltpu
from jax.experimental.pallas import tpu_sc as plsc

_D = 1024
_S = 2048
_NCORE = 2
_NSUB = 16
_NW = _NCORE * _NSUB
_RPW = _S // _NW
_CH = 16
_NCH = _RPW // _CH

_mesh = plsc.VectorSubcoreMesh(
    core_axis_name="c", subcore_axis_name="s",
    num_cores=_NCORE, num_subcores=_NSUB,
)


@functools.partial(
    pl.kernel,
    out_type=jax.ShapeDtypeStruct((_S + 1, 2, _D), jnp.float32),
    mesh=_mesh,
    scratch_types=[
        pltpu.VMEM((_CH, 2, _D), jnp.float32),
        pltpu.VMEM((_CH, 2, _D), jnp.float32),
        pltpu.VMEM((8,), jnp.int32),
        pltpu.VMEM((8,), jnp.int32),
        pltpu.VMEM((8, _D), jnp.float32),
        pltpu.VMEM((8, _D), jnp.float32),
        pltpu.VMEM((1, 2, _D), jnp.float32),
        pltpu.SemaphoreType.DMA,
        pltpu.SemaphoreType.DMA,
        pltpu.SemaphoreType.DMA,
        pltpu.SemaphoreType.DMA,
        pltpu.SemaphoreType.DMA,
    ],
)
def _sc_fused(bos_hbm, emb_hbm, idx_hbm, spe_hbm, pos_hbm, out_hbm,
              bufa, bufb, bi_v, pi_v, spe_v, pos_v, row_v,
              sem_ia, sem_ib, sem_oa, sem_ob, sem_g):
    c = lax.axis_index("c")
    s = lax.axis_index("s")
    w = s * _NCORE + c
    r0 = w * _RPW

    bufs = (bufa, bufb)
    sem_in = (sem_ia, sem_ib)
    sem_out = (sem_oa, sem_ob)

    def mk_in(t, b):
        return pltpu.make_async_copy(
            emb_hbm.at[b, pl.ds(r0 + t * _CH, _CH), :],
            bufs[t % 2].at[:, b, :], sem_in[t % 2])

    def mk_out(t):
        return pltpu.make_async_copy(
            bufs[t % 2], out_hbm.at[pl.ds(r0 + t * _CH, _CH), :, :],
            sem_out[t % 2])

    is_row_worker = w == _NW - 1

    def mk_bos(ref, dst):
        return pltpu.make_async_copy(ref, dst, sem_g)

    def mk_g1():
        return pltpu.make_async_copy(spe_hbm.at[bi_v], spe_v, sem_g)

    def mk_g2():
        return pltpu.make_async_copy(pos_hbm.at[pi_v], pos_v, sem_g)

    def mk_rw():
        return pltpu.make_async_copy(row_v, out_hbm.at[pl.ds(_S, 1), :, :], sem_g)

    mk_in(0, 0).start()
    mk_in(0, 1).start()
    mk_in(1, 0).start()
    mk_in(1, 1).start()

    @pl.when(is_row_worker)
    def _row0():
        mk_bos(bos_hbm, bi_v).start()
        mk_bos(idx_hbm, pi_v).start()

    for t in range(_NCH):
        mk_in(t, 0).wait()
        mk_in(t, 1).wait()
        mk_out(t).start()

        if t == 0:
            @pl.when(is_row_worker)
            def _row1():
                mk_bos(bos_hbm, bi_v).wait()
                mk_bos(idx_hbm, pi_v).wait()
                mk_g1().start()
                mk_g2().start()
        if t == 2:
            @pl.when(is_row_worker)
            def _row2():
                mk_g1().wait()
                mk_g2().wait()
                for j in range(_D // 16):
                    sl = pl.ds(16 * j, 16)
                    r = spe_v[0, sl] + pos_v[0, sl]
                    row_v[0, 0, sl] = r
                    row_v[0, 1, sl] = r
                mk_rw().start()

        mk_out(t).wait()
        if t + 2 < _NCH:
            mk_in(t + 2, 0).start()
            mk_in(t + 2, 1).start()

    @pl.when(is_row_worker)
    def _row3():
        mk_rw().wait()


def kernel(bos_token, embeds, idx, speech_emb, pos_emb):
    bos8 = jnp.broadcast_to(bos_token.reshape(1).astype(jnp.int32), (8,))
    idx8 = jnp.broadcast_to(idx.reshape(1).astype(jnp.int32), (8,))
    out2 = _sc_fused(bos8, embeds, idx8, speech_emb, pos_emb)
    return jnp.swapaxes(out2, 0, 1)
